# baseline (speedup 1.0000x reference)
import functools

import jax
import jax.numpy as jnp
from jax import lax
from jax.experimental import pallas as pl
from jax.experimental.pallas import tpu as pltpu
from jax.experimental.pallas import tpu_sc as plsc

N = 10000
N_PAD = 10240
E = 320000
D = 128
NC = 2
NS = 16
NW = NC * NS
C = 40
NCHUNK = E // (NW * C)
NPAIR = NCHUNK // 2
ROWS_PER_TILE = N_PAD // NS
NZ = ROWS_PER_TILE // C

_MESH = plsc.VectorSubcoreMesh(core_axis_name="c", subcore_axis_name="s")


@functools.partial(
    pl.kernel,
    mesh=_MESH,
    out_type=[
        jax.ShapeDtypeStruct((NC, N_PAD, D), jnp.float32),
        jax.ShapeDtypeStruct((NC, N_PAD, D), jnp.float32),
    ],
    scratch_types=[
        pltpu.VMEM((2, C), jnp.int32),
        pltpu.VMEM((2, C), jnp.int32),
        pltpu.VMEM((C, D), jnp.float32),
        pltpu.VMEM((C, D), jnp.float32),
        pltpu.SemaphoreType.DMA,
        pltpu.SemaphoreType.DMA,
        pltpu.SemaphoreType.DMA,
        pltpu.SemaphoreType.DMA,
        pltpu.VMEM_SHARED((N_PAD, D), jnp.float32),
    ],
)
def _sc_aggregate(feat_hbm, edges_hbm, zrow_hbm, ones_hbm,
                  acc_out, cnt_out,
                  idx_a, idx_b, rows_a, rows_b, sga, sgb, sia, sib, acc_sh):
    cid = lax.axis_index("c")
    sid = lax.axis_index("s")
    wid = sid * NC + cid
    base = sid * ROWS_PER_TILE
    j0 = wid * NCHUNK

    def zero_acc():
        pltpu.sync_copy(zrow_hbm, rows_a)
        for k in range(NZ):
            pltpu.make_async_copy(
                rows_a, acc_sh.at[pl.ds(base + k * C, C)], sga).start()
        for k in range(NZ):
            pltpu.make_async_copy(
                rows_a, acc_sh.at[pl.ds(base, C)], sga).wait()

    def copy_out(dst_hbm):
        def obody(k, carry):
            r = base + k * C
            pltpu.sync_copy(acc_sh.at[pl.ds(r, C)], rows_a)
            pltpu.sync_copy(rows_a, dst_hbm.at[cid, pl.ds(r, C)])
            return carry

        lax.fori_loop(0, NZ, obody, 0)

    zero_acc()
    plsc.subcore_barrier()

    pltpu.sync_copy(edges_hbm.at[j0], idx_a)
    pltpu.make_async_copy(feat_hbm.at[idx_a.at[1]], rows_a, sga).start()
    pltpu.make_async_copy(edges_hbm.at[j0 + 1], idx_b, sib).start()

    def body(k, carry):
        i = j0 + 2 * k
        pltpu.make_async_copy(edges_hbm.at[0], idx_b, sib).wait()
        pltpu.make_async_copy(feat_hbm.at[idx_b.at[1]], rows_b, sgb).start()
        pltpu.make_async_copy(feat_hbm.at[0:C], rows_a, sga).wait()
        pltpu.sync_copy(rows_a, acc_sh.at[idx_a.at[0]], add=True)
        pltpu.make_async_copy(edges_hbm.at[i + 2], idx_a, sia).start()
        pltpu.make_async_copy(edges_hbm.at[0], idx_a, sia).wait()
        pltpu.make_async_copy(feat_hbm.at[idx_a.at[1]], rows_a, sga).start()
        pltpu.make_async_copy(feat_hbm.at[0:C], rows_b, sgb).wait()
        pltpu.sync_copy(rows_b, acc_sh.at[idx_b.at[0]], add=True)
        pltpu.make_async_copy(edges_hbm.at[i + 3], idx_b, sib).start()
        return carry

    lax.fori_loop(0, NPAIR, body, 0)
    pltpu.make_async_copy(edges_hbm.at[0], idx_b, sib).wait()
    pltpu.make_async_copy(feat_hbm.at[0:C], rows_a, sga).wait()
    plsc.subcore_barrier()
    copy_out(acc_out)

    pltpu.sync_copy(ones_hbm, rows_b)
    plsc.subcore_barrier()

    pltpu.sync_copy(edges_hbm.at[j0], idx_a)
    pltpu.make_async_copy(edges_hbm.at[j0 + 1], idx_b, sib).start()

    def body2(k, carry):
        i = j0 + 2 * k
        pltpu.sync_copy(rows_b, acc_sh.at[idx_a.at[0]], add=True)
        pltpu.make_async_copy(edges_hbm.at[0], idx_b, sib).wait()
        pltpu.make_async_copy(edges_hbm.at[i + 2], idx_a, sia).start()
        pltpu.sync_copy(rows_b, acc_sh.at[idx_b.at[0]], add=True)
        pltpu.make_async_copy(edges_hbm.at[0], idx_a, sia).wait()
        pltpu.make_async_copy(edges_hbm.at[i + 3], idx_b, sib).start()
        return carry

    lax.fori_loop(0, NPAIR, body2, 0)
    pltpu.make_async_copy(edges_hbm.at[0], idx_b, sib).wait()
    plsc.subcore_barrier()
    copy_out(cnt_out)


_TC_R = 1024


def _tc_body(p_ref, c_ref, w_ref, b_ref, o_ref):
    agg = p_ref[0] + p_ref[1]
    cnt = (c_ref[0, :, 0:1] + c_ref[1, :, 0:1]
           - p_ref[0, :, 0:1] - p_ref[1, :, 0:1])
    h = agg / jnp.maximum(cnt, 1.0)
    o_ref[...] = lax.dot_general(
        h, w_ref[...], (((1,), (1,)), ((), ())),
        preferred_element_type=jnp.float32,
        precision=lax.Precision.HIGHEST,
    ) + b_ref[...]


def _tc_finish(acc, cnt, W, b2d):
    return pl.pallas_call(
        _tc_body,
        grid=(N_PAD // _TC_R,),
        in_specs=[
            pl.BlockSpec((NC, _TC_R, D), lambda i: (0, i, 0)),
            pl.BlockSpec((NC, _TC_R, D), lambda i: (0, i, 0)),
            pl.BlockSpec((D, D), lambda i: (0, 0)),
            pl.BlockSpec((1, D), lambda i: (0, 0)),
        ],
        out_specs=pl.BlockSpec((_TC_R, D), lambda i: (i, 0)),
        out_shape=jax.ShapeDtypeStruct((N, D), jnp.float32),
    )(acc, cnt, W, b2d)


def kernel(feat, edge_index, W, b):
    ei = edge_index.astype(jnp.int32)
    edges = ei[::-1].reshape(2, NW * NCHUNK, C).transpose(1, 0, 2)
    edges = jnp.pad(edges, ((0, 2), (0, 0), (0, 0)))
    zrow = jnp.zeros((C, D), jnp.float32)
    ones = jnp.ones((C, D), jnp.float32)
    acc, cnt = _sc_aggregate(feat, edges, zrow, ones)
    return _tc_finish(acc, cnt, W, b.reshape(1, D))
